# baseline (speedup 1.0000x reference)
import jax
import jax.numpy as jnp
from jax import lax
from jax.experimental import pallas as pl
from jax.experimental.pallas import tpu as pltpu
from jax.experimental.pallas import tpu_sc as plsc

VOCAB = 100000
EMBED_DIM = 64
BATCH = 1024
CTX = 20

_NC = 2
_NS = 16
_NW = _NC * _NS
_BPW = BATCH // _NW
_EPW = _BPW * CTX
_GCHUNK = 128
_NCHUNK = _EPW // _GCHUNK
_PD = 128


def _sc_pool_body(idx_hbm, pairs_hbm, out_hbm, idx_v, pair_v, rows_v,
                  pooled_v, sem):
  wid = lax.axis_index("s") * _NC + lax.axis_index("c")
  ebase = wid * _EPW

  pltpu.sync_copy(idx_hbm.at[pl.ds(ebase, _EPW)], idx_v.at[pl.ds(0, _EPW)])

  for c in range(_EPW // 16):
    pair_v[pl.ds(c * 16, 16)] = jax.lax.shift_right_logical(
        idx_v[pl.ds(c * 16, 16)], 1)

  copies = []
  for j in range(_NCHUNK):
    copies.append(
        pltpu.async_copy(
            pairs_hbm.at[pair_v.at[pl.ds(j * _GCHUNK, _GCHUNK)]],
            rows_v.at[pl.ds(j * _GCHUNK, _GCHUNK)],
            sem,
        )
    )
  for c in copies:
    c.wait()

  scale = jnp.float32(1.0 / CTX)

  def body(b, _):
    e0 = b * CTX
    offs = []
    for j in range(CTX):
      v = idx_v[pl.ds(e0 + j, 16)]
      offs.append((v[0] & 1) * EMBED_DIM)
    for d in range(EMBED_DIM // 16):
      acc = rows_v[e0, pl.ds(offs[0] + d * 16, 16)]
      for j in range(1, CTX):
        acc = acc + rows_v[e0 + j, pl.ds(offs[j] + d * 16, 16)]
      pooled_v[b, pl.ds(d * 16, 16)] = acc * scale
    return 0

  lax.fori_loop(0, _BPW, body, 0)

  pltpu.sync_copy(pooled_v, out_hbm.at[pl.ds(wid * _BPW, _BPW)])


def _sc_pool(idx_flat, emb_pairs):
  mesh = plsc.VectorSubcoreMesh(core_axis_name="c", subcore_axis_name="s")
  return pl.kernel(
      _sc_pool_body,
      out_type=jax.ShapeDtypeStruct((BATCH, _PD), jnp.float32),
      mesh=mesh,
      scratch_types=[
          pltpu.VMEM((_EPW + 16,), jnp.int32),
          pltpu.VMEM((_EPW,), jnp.int32),
          pltpu.VMEM((_EPW, _PD), jnp.float32),
          pltpu.VMEM((_BPW, _PD), jnp.float32),
          pltpu.SemaphoreType.DMA,
      ],
  )(idx_flat, emb_pairs)


_NPAIR = VOCAB // 2
_PPW = 1568
_PCH = 256


def _sc_repack_body(table_hbm, pairs_hbm, in_v, out_v, sem):
  wid = lax.axis_index("s") * _NC + lax.axis_index("c")
  p0 = wid * _PPW
  pend = jnp.minimum(p0 + _PPW, _NPAIR)
  nch = (pend - p0 + _PCH - 1) // _PCH

  def chunk(k, _):
    p = jnp.minimum(p0 + k * _PCH, pend - _PCH)
    cin = pltpu.make_async_copy(
        table_hbm.at[pl.ds(p * 2, _PCH * 2)], in_v, sem)
    cin.start()
    cin.wait()
    def pair(q, _):
      for h in range(2):
        for d in range(EMBED_DIM // 16):
          out_v[q, pl.ds(h * EMBED_DIM + d * 16, 16)] = (
              in_v[2 * q + h, pl.ds(d * 16, 16)])
      return 0

    lax.fori_loop(0, _PCH, pair, 0)
    cout = pltpu.make_async_copy(out_v, pairs_hbm.at[pl.ds(p, _PCH)], sem)
    cout.start()
    cout.wait()
    return 0

  lax.fori_loop(0, nch, chunk, 0)


def _sc_repack(emb_table):
  mesh = plsc.VectorSubcoreMesh(core_axis_name="c", subcore_axis_name="s")
  return pl.kernel(
      _sc_repack_body,
      out_type=jax.ShapeDtypeStruct((_NPAIR, 2 * EMBED_DIM), jnp.float32),
      mesh=mesh,
      scratch_types=[
          pltpu.VMEM((_PCH * 2, EMBED_DIM), jnp.float32),
          pltpu.VMEM((_PCH, 2 * EMBED_DIM), jnp.float32),
          pltpu.SemaphoreType.DMA,
      ],
  )(emb_table)


_TV = 2048


def _proj_body(xt_ref, wt_ref, b_ref, out_ref):
  bias_col = b_ref[...].reshape(_TV, 1)
  out_ref[...] = (
      lax.dot_general(
          wt_ref[...],
          xt_ref[...],
          (((0,), (0,)), ((), ())),
          preferred_element_type=jnp.float32,
      )
      + bias_col
  )


def _projection(pooled_t, lin_w_t, lin_b_row):
  grid = (pl.cdiv(VOCAB, _TV),)
  return pl.pallas_call(
      _proj_body,
      grid=grid,
      in_specs=[
          pl.BlockSpec((EMBED_DIM, BATCH), lambda i: (0, 0)),
          pl.BlockSpec((EMBED_DIM, _TV), lambda i: (0, i)),
          pl.BlockSpec((1, _TV), lambda i: (0, i)),
      ],
      out_specs=pl.BlockSpec((_TV, BATCH), lambda i: (i, 0)),
      out_shape=jax.ShapeDtypeStruct((VOCAB, BATCH), jnp.float32),
      compiler_params=pltpu.CompilerParams(
          dimension_semantics=("arbitrary",)),
  )(pooled_t, lin_w_t, lin_b_row)


@jax.jit
def kernel(inputs_, emb_table, lin_w, lin_b):
  idx_flat = inputs_.reshape(-1).astype(jnp.int32)
  emb_pairs = _sc_repack(emb_table)
  pooled_pad = _sc_pool(idx_flat, emb_pairs)
  pooled_t = pooled_pad[:, :EMBED_DIM].T
  out_t = _projection(pooled_t, lin_w.T, lin_b.reshape(1, VOCAB))
  return out_t.T


# speedup vs baseline: 1.2773x; 1.2773x over previous
import jax
import jax.numpy as jnp
from jax import lax
from jax.experimental import pallas as pl
from jax.experimental.pallas import tpu as pltpu
from jax.experimental.pallas import tpu_sc as plsc

VOCAB = 100000
EMBED_DIM = 64
BATCH = 1024
CTX = 20

_NC = 2
_NS = 16
_NW = _NC * _NS
_BPW = BATCH // _NW
_EPW = _BPW * CTX
_GCHUNK = 128
_NCHUNK = _EPW // _GCHUNK
_PD = 128


def _sc_pool_body(idx_hbm, pairs_hbm, out_hbm, idx_v, pair_v, rows_v,
                  pooled_v, sem):
  wid = lax.axis_index("s") * _NC + lax.axis_index("c")
  ebase = wid * _EPW

  pltpu.sync_copy(idx_hbm.at[pl.ds(ebase, _EPW)], idx_v.at[pl.ds(0, _EPW)])

  for c in range(_EPW // 16):
    pair_v[pl.ds(c * 16, 16)] = jax.lax.shift_right_logical(
        idx_v[pl.ds(c * 16, 16)], 1)

  copies = []
  for j in range(_NCHUNK):
    copies.append(
        pltpu.async_copy(
            pairs_hbm.at[pair_v.at[pl.ds(j * _GCHUNK, _GCHUNK)]],
            rows_v.at[pl.ds(j * _GCHUNK, _GCHUNK)],
            sem,
        )
    )
  for c in copies:
    c.wait()

  scale = jnp.float32(1.0 / CTX)

  def body(b, _):
    e0 = b * CTX
    offs = []
    for j in range(CTX):
      v = idx_v[pl.ds(e0 + j, 16)]
      offs.append((v[0] & 1) * EMBED_DIM)
    for d in range(EMBED_DIM // 16):
      acc = rows_v[e0, pl.ds(offs[0] + d * 16, 16)]
      for j in range(1, CTX):
        acc = acc + rows_v[e0 + j, pl.ds(offs[j] + d * 16, 16)]
      pooled_v[b, pl.ds(d * 16, 16)] = acc * scale
    return 0

  lax.fori_loop(0, _BPW, body, 0)

  pltpu.sync_copy(pooled_v, out_hbm.at[pl.ds(wid * _BPW, _BPW)])


def _sc_pool(idx_flat, emb_pairs):
  mesh = plsc.VectorSubcoreMesh(core_axis_name="c", subcore_axis_name="s")
  return pl.kernel(
      _sc_pool_body,
      out_type=jax.ShapeDtypeStruct((BATCH, _PD), jnp.float32),
      mesh=mesh,
      scratch_types=[
          pltpu.VMEM((_EPW + 16,), jnp.int32),
          pltpu.VMEM((_EPW,), jnp.int32),
          pltpu.VMEM((_EPW, _PD), jnp.float32),
          pltpu.VMEM((_BPW, _PD), jnp.float32),
          pltpu.SemaphoreType.DMA,
      ],
  )(idx_flat, emb_pairs)


_NPAIR = VOCAB // 2
_PPW = 1568
_PCH = 256


def _sc_repack_body(table_hbm, pairs_hbm, in_v, out_v, sem):
  wid = lax.axis_index("s") * _NC + lax.axis_index("c")
  p0 = wid * _PPW
  pend = jnp.minimum(p0 + _PPW, _NPAIR)
  nch = (pend - p0 + _PCH - 1) // _PCH

  def chunk(k, _):
    p = jnp.minimum(p0 + k * _PCH, pend - _PCH)
    cin = pltpu.make_async_copy(
        table_hbm.at[pl.ds(p * 2, _PCH * 2)], in_v, sem)
    cin.start()
    cin.wait()
    def pair(q, _):
      for h in range(2):
        for d in range(EMBED_DIM // 16):
          out_v[q, pl.ds(h * EMBED_DIM + d * 16, 16)] = (
              in_v[2 * q + h, pl.ds(d * 16, 16)])
      return 0

    lax.fori_loop(0, _PCH, pair, 0)
    cout = pltpu.make_async_copy(out_v, pairs_hbm.at[pl.ds(p, _PCH)], sem)
    cout.start()
    cout.wait()
    return 0

  lax.fori_loop(0, nch, chunk, 0)


def _sc_repack(emb_table):
  mesh = plsc.VectorSubcoreMesh(core_axis_name="c", subcore_axis_name="s")
  return pl.kernel(
      _sc_repack_body,
      out_type=jax.ShapeDtypeStruct((_NPAIR, 2 * EMBED_DIM), jnp.float32),
      mesh=mesh,
      scratch_types=[
          pltpu.VMEM((_PCH * 2, EMBED_DIM), jnp.float32),
          pltpu.VMEM((_PCH, 2 * EMBED_DIM), jnp.float32),
          pltpu.SemaphoreType.DMA,
      ],
  )(emb_table)


_TV = 4096


def _proj_body(xt_ref, wt_ref, b_ref, out_ref):
  bias_col = b_ref[...].reshape(_TV, 1)
  out_ref[...] = (
      lax.dot_general(
          wt_ref[...],
          xt_ref[...],
          (((0,), (0,)), ((), ())),
          preferred_element_type=jnp.float32,
      )
      + bias_col
  )


def _projection(pooled_t, lin_w_t, lin_b_row):
  grid = (pl.cdiv(VOCAB, _TV),)
  return pl.pallas_call(
      _proj_body,
      grid=grid,
      in_specs=[
          pl.BlockSpec((EMBED_DIM, BATCH), lambda i: (0, 0)),
          pl.BlockSpec((EMBED_DIM, _TV), lambda i: (0, i)),
          pl.BlockSpec((1, _TV), lambda i: (0, i)),
      ],
      out_specs=pl.BlockSpec((_TV, BATCH), lambda i: (i, 0)),
      out_shape=jax.ShapeDtypeStruct((VOCAB, BATCH), jnp.float32),
      compiler_params=pltpu.CompilerParams(
          dimension_semantics=("arbitrary",)),
  )(pooled_t, lin_w_t, lin_b_row)


@jax.jit
def kernel(inputs_, emb_table, lin_w, lin_b):
  idx_flat = inputs_.reshape(-1).astype(jnp.int32)
  emb_pairs = emb_table.reshape(VOCAB // 2, 2 * EMBED_DIM)
  pooled_pad = _sc_pool(idx_flat, emb_pairs)
  pooled_t = pooled_pad[:, :EMBED_DIM].T
  out_t = _projection(pooled_t, lin_w.T, lin_b.reshape(1, VOCAB))
  return out_t.T
